# baseline (speedup 1.0000x reference)
import functools

import jax
import jax.numpy as jnp
from jax import lax
from jax.experimental import pallas as pl
from jax.experimental.pallas import tpu as pltpu
from jax.experimental.pallas import tpu_sc as plsc

RANK = 16
LANES = 16
NUM_WORKERS = 32
CHUNK = 800


def _tatd_kernel(nnz, ndim):
    num_chunks = nnz // CHUNK
    assert num_chunks * CHUNK == nnz and CHUNK % LANES == 0
    groups = CHUNK // LANES
    base_n = num_chunks // NUM_WORKERS
    rem = num_chunks % NUM_WORKERS
    max_n = base_n + (1 if rem else 0)
    outer_iters = (max_n + 1) // 2

    mesh = plsc.VectorSubcoreMesh(core_axis_name="c", subcore_axis_name="s")

    @functools.partial(
        pl.kernel,
        mesh=mesh,
        compiler_params=pltpu.CompilerParams(
            needs_layout_passes=False, use_tc_tiling_on_sc=False),
        out_type=jax.ShapeDtypeStruct((nnz,), jnp.float32),
        scratch_types=[
            pltpu.VMEM((CHUNK,), jnp.int32),
            pltpu.VMEM((CHUNK,), jnp.int32),
            pltpu.VMEM((CHUNK,), jnp.int32),
            pltpu.VMEM((CHUNK, RANK), jnp.float32),
            pltpu.VMEM((CHUNK, RANK), jnp.float32),
            pltpu.VMEM((CHUNK, RANK), jnp.float32),
            pltpu.VMEM((CHUNK, RANK), jnp.float32),
            pltpu.VMEM((CHUNK, RANK), jnp.float32),
            pltpu.VMEM((CHUNK, RANK), jnp.float32),
            pltpu.VMEM((CHUNK,), jnp.float32),
            pltpu.VMEM((CHUNK,), jnp.float32),
            pltpu.SemaphoreType.DMA,
            pltpu.SemaphoreType.DMA,
            pltpu.SemaphoreType.DMA,
            pltpu.SemaphoreType.DMA,
        ],
    )
    def k(idx_hbm, f0_hbm, f1_hbm, f2_hbm, out_hbm,
          idx0_v, idx1_v, idx2_v,
          r0a, r1a, r2a, r0b, r1b, r2b,
          out_a, out_b,
          sem_ga, sem_gb, sem_oa, sem_ob):
        wid = lax.axis_index("s") * 2 + lax.axis_index("c")
        extra = jnp.minimum(wid, rem)
        lo = wid * base_n + extra
        n = base_n + jnp.where(wid < rem, 1, 0)
        lane = lax.iota(jnp.int32, LANES)
        rows = ((r0a, r1a, r2a), (r0b, r1b, r2b))
        outs = (out_a, out_b)
        sems_g = (sem_ga, sem_gb)
        sems_o = (sem_oa, sem_ob)
        fs = (f0_hbm, f1_hbm, f2_hbm)
        idxs = (idx0_v, idx1_v, idx2_v)

        def load_idx(chunk_id):
            base = chunk_id * CHUNK
            for m in range(3):
                pltpu.sync_copy(idx_hbm.at[m, pl.ds(base, CHUNK)],
                                idxs[m])

        def issue_gathers(b):
            for m in range(3):
                pltpu.async_copy(fs[m].at[idxs[m]], rows[b][m], sems_g[b])

        def wait_gathers(b):
            for m in range(3):
                pltpu.make_async_copy(fs[m].at[idxs[m]], rows[b][m],
                                      sems_g[b]).wait()

        def compute(b):
            r0, r1, r2 = rows[b]
            out_v = outs[b]

            @plsc.parallel_loop(0, groups)
            def group_body(g):
                row_ids = g * LANES + lane
                acc = jnp.zeros((LANES,), jnp.float32)
                for r in range(RANK):
                    col = jnp.full((LANES,), r, jnp.int32)
                    v0 = plsc.load_gather(r0, [row_ids, col])
                    v1 = plsc.load_gather(r1, [row_ids, col])
                    v2 = plsc.load_gather(r2, [row_ids, col])
                    acc = acc + v0 * v1 * v2
                out_v[pl.ds(g * LANES, LANES)] = acc

        def issue_out(kk, b):
            base = (lo + kk) * CHUNK
            pltpu.async_copy(outs[b], out_hbm.at[pl.ds(base, CHUNK)],
                             sems_o[b])

        def wait_out(b):
            pltpu.make_async_copy(outs[b], out_hbm.at[pl.ds(0, CHUNK)],
                                  sems_o[b]).wait()

        load_idx(lo)
        issue_gathers(0)

        def body(kk, b):
            wait_gathers(b)

            @pl.when(kk + 1 < n)
            def _():
                load_idx(lo + kk + 1)
                issue_gathers(1 - b)

            @pl.when(kk >= 2)
            def _():
                wait_out(b)

            compute(b)
            issue_out(kk, b)

        def outer(i, _):
            kk = i * 2

            @pl.when(kk < n)
            def _():
                body(kk, 0)

            @pl.when(kk + 1 < n)
            def _():
                body(kk + 1, 1)

            return 0

        lax.fori_loop(0, outer_iters, outer, 0)

        wait_out((0))
        wait_out((1))

    return k


DETILE_B = 800


def _detile_kernel(ndim):
    pieces = ndim // DETILE_B
    assert pieces * DETILE_B == ndim
    outer = (pieces + NUM_WORKERS - 1) // NUM_WORKERS

    mesh = plsc.VectorSubcoreMesh(core_axis_name="c", subcore_axis_name="s")

    @functools.partial(
        pl.kernel,
        mesh=mesh,
        compiler_params=pltpu.CompilerParams(
            needs_layout_passes=False, use_tc_tiling_on_sc=True),
        out_type=[jax.ShapeDtypeStruct((ndim * RANK,), jnp.float32)] * 3,
        scratch_types=[
            pltpu.VMEM((DETILE_B, RANK), jnp.float32),
            pltpu.VMEM((DETILE_B * RANK,), jnp.float32),
        ],
    )
    def k(f0_hbm, f1_hbm, f2_hbm, g0_hbm, g1_hbm, g2_hbm, buf2d, buf1d):
        wid = lax.axis_index("s") * 2 + lax.axis_index("c")
        fs = (f0_hbm, f1_hbm, f2_hbm)
        gs = (g0_hbm, g1_hbm, g2_hbm)

        def piece(p, _):
            pid = p * NUM_WORKERS + wid

            @pl.when(pid < pieces)
            def _():
                row0 = pid * DETILE_B
                for m in range(3):
                    pltpu.sync_copy(fs[m].at[pl.ds(row0, DETILE_B), :], buf2d)

                    @plsc.parallel_loop(0, DETILE_B)
                    def move(i):
                        buf1d[pl.ds(i * RANK, RANK)] = buf2d[i, :]

                    pltpu.sync_copy(
                        buf1d, gs[m].at[pl.ds(row0 * RANK, DETILE_B * RANK)])

            return 0

        lax.fori_loop(0, outer, piece, 0)

    return k


def kernel(indices_list, f0, f1, f2):
    nnz = indices_list.shape[1]
    ndim = f0.shape[0]
    g0, g1, g2 = _detile_kernel(ndim)(f0, f1, f2)
    return _tatd_kernel(nnz, ndim)(
        indices_list.astype(jnp.int32),
        g0.reshape(ndim, RANK), g1.reshape(ndim, RANK), g2.reshape(ndim, RANK))


# speedup vs baseline: 1.5162x; 1.5162x over previous
import functools

import jax
import jax.numpy as jnp
from jax import lax
from jax.experimental import pallas as pl
from jax.experimental.pallas import tpu as pltpu
from jax.experimental.pallas import tpu_sc as plsc

RANK = 16
LANES = 16
NUM_WORKERS = 32
CHUNK = 1000


def _tatd_kernel(nnz, ndim):
    num_chunks = nnz // CHUNK
    assert num_chunks * CHUNK == nnz
    groups = (CHUNK + LANES - 1) // LANES
    base_n = num_chunks // NUM_WORKERS
    rem = num_chunks % NUM_WORKERS
    max_n = base_n + (1 if rem else 0)
    outer_iters = (max_n + 1) // 2

    mesh = plsc.VectorSubcoreMesh(core_axis_name="c", subcore_axis_name="s")

    @functools.partial(
        pl.kernel,
        mesh=mesh,
        compiler_params=pltpu.CompilerParams(
            needs_layout_passes=False, use_tc_tiling_on_sc=False),
        out_type=jax.ShapeDtypeStruct((nnz,), jnp.float32),
        scratch_types=[
            pltpu.VMEM((CHUNK,), jnp.int32),
            pltpu.VMEM((CHUNK,), jnp.int32),
            pltpu.VMEM((CHUNK,), jnp.int32),
            pltpu.VMEM((CHUNK,), jnp.int32),
            pltpu.VMEM((CHUNK,), jnp.int32),
            pltpu.VMEM((CHUNK,), jnp.int32),
            pltpu.VMEM((CHUNK, RANK), jnp.float32),
            pltpu.VMEM((CHUNK, RANK), jnp.float32),
            pltpu.VMEM((CHUNK, RANK), jnp.float32),
            pltpu.VMEM((CHUNK, RANK), jnp.float32),
            pltpu.VMEM((CHUNK, RANK), jnp.float32),
            pltpu.VMEM((CHUNK, RANK), jnp.float32),
            pltpu.VMEM((CHUNK,), jnp.float32),
            pltpu.VMEM((CHUNK,), jnp.float32),
            pltpu.SemaphoreType.DMA,
            pltpu.SemaphoreType.DMA,
            pltpu.SemaphoreType.DMA,
            pltpu.SemaphoreType.DMA,
            pltpu.SemaphoreType.DMA,
            pltpu.SemaphoreType.DMA,
        ],
    )
    def k(i0_hbm, i1_hbm, i2_hbm, f0_hbm, f1_hbm, f2_hbm, out_hbm,
          i0a, i1a, i2a, i0b, i1b, i2b,
          r0a, r1a, r2a, r0b, r1b, r2b,
          out_a, out_b,
          sem_ia, sem_ib, sem_ga, sem_gb, sem_oa, sem_ob):
        wid = lax.axis_index("s") * 2 + lax.axis_index("c")
        extra = jnp.minimum(wid, rem)
        lo = wid * base_n + extra
        n = base_n + jnp.where(wid < rem, 1, 0)
        lane = lax.iota(jnp.int32, LANES)
        idxs = ((i0a, i1a, i2a), (i0b, i1b, i2b))
        rows = ((r0a, r1a, r2a), (r0b, r1b, r2b))
        outs = (out_a, out_b)
        sems_i = (sem_ia, sem_ib)
        sems_g = (sem_ga, sem_gb)
        sems_o = (sem_oa, sem_ob)
        fs = (f0_hbm, f1_hbm, f2_hbm)
        is_hbm = (i0_hbm, i1_hbm, i2_hbm)

        def issue_idx(chunk_id, b):
            base = chunk_id * CHUNK
            for m in range(3):
                pltpu.async_copy(is_hbm[m].at[pl.ds(base, CHUNK)],
                                 idxs[b][m], sems_i[b])

        def wait_idx(b):
            for m in range(3):
                pltpu.make_async_copy(is_hbm[m].at[pl.ds(0, CHUNK)],
                                      idxs[b][m], sems_i[b]).wait()

        def issue_gathers(b):
            for m in range(3):
                pltpu.async_copy(fs[m].at[idxs[b][m]], rows[b][m], sems_g[b])

        def wait_gathers(b):
            for m in range(3):
                pltpu.make_async_copy(fs[m].at[idxs[b][m]], rows[b][m],
                                      sems_g[b]).wait()

        def compute(b):
            r0, r1, r2 = rows[b]
            out_v = outs[b]

            @plsc.parallel_loop(0, groups)
            def group_body(g):
                off = jnp.minimum(g * LANES, CHUNK - LANES)
                row_ids = off + lane
                acc = jnp.zeros((LANES,), jnp.float32)
                for r in range(RANK):
                    col = jnp.full((LANES,), r, jnp.int32)
                    v0 = plsc.load_gather(r0, [row_ids, col])
                    v1 = plsc.load_gather(r1, [row_ids, col])
                    v2 = plsc.load_gather(r2, [row_ids, col])
                    acc = acc + v0 * v1 * v2
                out_v[pl.ds(off, LANES)] = acc

        def issue_out(kk, b):
            base = (lo + kk) * CHUNK
            pltpu.async_copy(outs[b], out_hbm.at[pl.ds(base, CHUNK)],
                             sems_o[b])

        def wait_out(b):
            pltpu.make_async_copy(outs[b], out_hbm.at[pl.ds(0, CHUNK)],
                                  sems_o[b]).wait()

        issue_idx(lo, 0)
        wait_idx(0)
        issue_gathers(0)
        issue_idx(lo + 1, 1)

        def body(kk, b):
            wait_gathers(b)

            @pl.when(kk + 1 < n)
            def _():
                wait_idx(1 - b)
                issue_gathers(1 - b)

            @pl.when(kk + 2 < n)
            def _():
                issue_idx(lo + kk + 2, b)

            @pl.when(kk >= 2)
            def _():
                wait_out(b)

            compute(b)
            issue_out(kk, b)

        def outer(i, _):
            kk = i * 2

            @pl.when(kk < n)
            def _():
                body(kk, 0)

            @pl.when(kk + 1 < n)
            def _():
                body(kk + 1, 1)

            return 0

        lax.fori_loop(0, outer_iters, outer, 0)

        wait_out(0)
        wait_out(1)

    return k


def kernel(indices_list, f0, f1, f2):
    nnz = indices_list.shape[1]
    ndim = f0.shape[0]
    idx = indices_list.astype(jnp.int32)
    return _tatd_kernel(nnz, ndim)(idx[0], idx[1], idx[2], f0, f1, f2)


# speedup vs baseline: 1.5668x; 1.0334x over previous
import functools

import jax
import jax.numpy as jnp
from jax import lax
from jax.experimental import pallas as pl
from jax.experimental.pallas import tpu as pltpu
from jax.experimental.pallas import tpu_sc as plsc

RANK = 16
LANES = 16
NUM_WORKERS = 32
CHUNK = 1000

_COMPILER_PARAMS = pltpu.CompilerParams(
    needs_layout_passes=False, use_tc_tiling_on_sc=False)


def _worker_range(wid, num_chunks):
    base_n = num_chunks // NUM_WORKERS
    rem = num_chunks % NUM_WORKERS
    extra = jnp.minimum(wid, rem)
    lo = wid * base_n + extra
    n = base_n + jnp.where(wid < rem, 1, 0)
    return lo, n


def _outer_iters(num_chunks):
    max_n = num_chunks // NUM_WORKERS + (1 if num_chunks % NUM_WORKERS else 0)
    return (max_n + 1) // 2


def _mul_kernel(nnz, ndim):
    num_chunks = nnz // CHUNK
    assert num_chunks * CHUNK == nnz

    mesh = plsc.VectorSubcoreMesh(core_axis_name="c", subcore_axis_name="s")

    @functools.partial(
        pl.kernel,
        mesh=mesh,
        compiler_params=_COMPILER_PARAMS,
        out_type=jax.ShapeDtypeStruct((nnz, RANK), jnp.float32),
        scratch_types=[
            pltpu.VMEM((CHUNK,), jnp.int32),
            pltpu.VMEM((CHUNK,), jnp.int32),
            pltpu.VMEM((CHUNK,), jnp.int32),
            pltpu.VMEM((CHUNK,), jnp.int32),
            pltpu.VMEM((CHUNK, RANK), jnp.float32),
            pltpu.VMEM((CHUNK, RANK), jnp.float32),
            pltpu.VMEM((CHUNK, RANK), jnp.float32),
            pltpu.VMEM((CHUNK, RANK), jnp.float32),
            pltpu.VMEM((CHUNK, RANK), jnp.float32),
            pltpu.VMEM((CHUNK, RANK), jnp.float32),
            pltpu.SemaphoreType.DMA,
            pltpu.SemaphoreType.DMA,
            pltpu.SemaphoreType.DMA,
            pltpu.SemaphoreType.DMA,
            pltpu.SemaphoreType.DMA,
            pltpu.SemaphoreType.DMA,
        ],
    )
    def k(i0_hbm, i1_hbm, f0_hbm, f1_hbm, p_hbm,
          i0a, i1a, i0b, i1b,
          r0a, r1a, r0b, r1b, pa, pb,
          sem_ia, sem_ib, sem_ga, sem_gb, sem_oa, sem_ob):
        wid = lax.axis_index("s") * 2 + lax.axis_index("c")
        lo, n = _worker_range(wid, num_chunks)
        idxs = ((i0a, i1a), (i0b, i1b))
        rows = ((r0a, r1a), (r0b, r1b))
        outs = (pa, pb)
        sems_i = (sem_ia, sem_ib)
        sems_g = (sem_ga, sem_gb)
        sems_o = (sem_oa, sem_ob)
        fs = (f0_hbm, f1_hbm)
        is_hbm = (i0_hbm, i1_hbm)

        def issue_idx(chunk_id, b):
            base = chunk_id * CHUNK
            for m in range(2):
                pltpu.async_copy(is_hbm[m].at[pl.ds(base, CHUNK)],
                                 idxs[b][m], sems_i[b])

        def wait_idx(b):
            for m in range(2):
                pltpu.make_async_copy(is_hbm[m].at[pl.ds(0, CHUNK)],
                                      idxs[b][m], sems_i[b]).wait()

        def issue_gathers(b):
            for m in range(2):
                pltpu.async_copy(fs[m].at[idxs[b][m]], rows[b][m], sems_g[b])

        def wait_gathers(b):
            for m in range(2):
                pltpu.make_async_copy(fs[m].at[idxs[b][m]], rows[b][m],
                                      sems_g[b]).wait()

        def compute(b):
            r0, r1 = rows[b]
            p_v = outs[b]

            @plsc.parallel_loop(0, CHUNK)
            def row_body(c):
                p_v[c, :] = r0[c, :] * r1[c, :]

        def issue_out(kk, b):
            base = (lo + kk) * CHUNK
            pltpu.async_copy(outs[b], p_hbm.at[pl.ds(base, CHUNK), :],
                             sems_o[b])

        def wait_out(b):
            pltpu.make_async_copy(outs[b], p_hbm.at[pl.ds(0, CHUNK), :],
                                  sems_o[b]).wait()

        issue_idx(lo, 0)
        wait_idx(0)
        issue_gathers(0)
        issue_idx(lo + 1, 1)

        def body(kk, b):
            wait_gathers(b)

            @pl.when(kk + 1 < n)
            def _():
                wait_idx(1 - b)
                issue_gathers(1 - b)

            @pl.when(kk + 2 < n)
            def _():
                issue_idx(lo + kk + 2, b)

            @pl.when(kk >= 2)
            def _():
                wait_out(b)

            compute(b)
            issue_out(kk, b)

        def outer(i, _):
            kk = i * 2

            @pl.when(kk < n)
            def _():
                body(kk, 0)

            @pl.when(kk + 1 < n)
            def _():
                body(kk + 1, 1)

            return 0

        lax.fori_loop(0, _outer_iters(num_chunks), outer, 0)
        wait_out(0)
        wait_out(1)

    return k


def _reduce_kernel(nnz, ndim):
    num_chunks = nnz // CHUNK
    groups = (CHUNK + LANES - 1) // LANES

    mesh = plsc.VectorSubcoreMesh(core_axis_name="c", subcore_axis_name="s")

    @functools.partial(
        pl.kernel,
        mesh=mesh,
        compiler_params=_COMPILER_PARAMS,
        out_type=jax.ShapeDtypeStruct((nnz,), jnp.float32),
        scratch_types=[
            pltpu.VMEM((CHUNK,), jnp.int32),
            pltpu.VMEM((CHUNK,), jnp.int32),
            pltpu.VMEM((CHUNK, RANK), jnp.float32),
            pltpu.VMEM((CHUNK, RANK), jnp.float32),
            pltpu.VMEM((CHUNK, RANK), jnp.float32),
            pltpu.VMEM((CHUNK, RANK), jnp.float32),
            pltpu.VMEM((CHUNK,), jnp.float32),
            pltpu.VMEM((CHUNK,), jnp.float32),
            pltpu.SemaphoreType.DMA,
            pltpu.SemaphoreType.DMA,
            pltpu.SemaphoreType.DMA,
            pltpu.SemaphoreType.DMA,
            pltpu.SemaphoreType.DMA,
            pltpu.SemaphoreType.DMA,
        ],
    )
    def k(i2_hbm, f2_hbm, p_hbm, out_hbm,
          i2a, i2b, r2a, r2b, pva, pvb, out_a, out_b,
          sem_ia, sem_ib, sem_ga, sem_gb, sem_oa, sem_ob):
        wid = lax.axis_index("s") * 2 + lax.axis_index("c")
        lo, n = _worker_range(wid, num_chunks)
        lane = lax.iota(jnp.int32, LANES)
        idxs = (i2a, i2b)
        rows = (r2a, r2b)
        pvs = (pva, pvb)
        outs = (out_a, out_b)
        sems_i = (sem_ia, sem_ib)
        sems_g = (sem_ga, sem_gb)
        sems_o = (sem_oa, sem_ob)

        def issue_idx(chunk_id, b):
            base = chunk_id * CHUNK
            pltpu.async_copy(i2_hbm.at[pl.ds(base, CHUNK)], idxs[b],
                             sems_i[b])

        def wait_idx(b):
            pltpu.make_async_copy(i2_hbm.at[pl.ds(0, CHUNK)], idxs[b],
                                  sems_i[b]).wait()

        def issue_gathers(chunk_id, b):
            base = chunk_id * CHUNK
            pltpu.async_copy(f2_hbm.at[idxs[b]], rows[b], sems_g[b])
            pltpu.async_copy(p_hbm.at[pl.ds(base, CHUNK), :], pvs[b],
                             sems_g[b])

        def wait_gathers(b):
            pltpu.make_async_copy(f2_hbm.at[idxs[b]], rows[b],
                                  sems_g[b]).wait()
            pltpu.make_async_copy(p_hbm.at[pl.ds(0, CHUNK), :], pvs[b],
                                  sems_g[b]).wait()

        def compute(b):
            r2, p_v, out_v = rows[b], pvs[b], outs[b]

            @plsc.parallel_loop(0, groups)
            def group_body(g):
                off = jnp.minimum(g * LANES, CHUNK - LANES)
                row_ids = off + lane
                acc = jnp.zeros((LANES,), jnp.float32)
                for r in range(RANK):
                    col = jnp.full((LANES,), r, jnp.int32)
                    vp = plsc.load_gather(p_v, [row_ids, col])
                    v2 = plsc.load_gather(r2, [row_ids, col])
                    acc = acc + vp * v2
                out_v[pl.ds(off, LANES)] = acc

        def issue_out(kk, b):
            base = (lo + kk) * CHUNK
            pltpu.async_copy(outs[b], out_hbm.at[pl.ds(base, CHUNK)],
                             sems_o[b])

        def wait_out(b):
            pltpu.make_async_copy(outs[b], out_hbm.at[pl.ds(0, CHUNK)],
                                  sems_o[b]).wait()

        issue_idx(lo, 0)
        wait_idx(0)
        issue_gathers(lo, 0)
        issue_idx(lo + 1, 1)

        def body(kk, b):
            wait_gathers(b)

            @pl.when(kk + 1 < n)
            def _():
                wait_idx(1 - b)
                issue_gathers(lo + kk + 1, 1 - b)

            @pl.when(kk + 2 < n)
            def _():
                issue_idx(lo + kk + 2, b)

            @pl.when(kk >= 2)
            def _():
                wait_out(b)

            compute(b)
            issue_out(kk, b)

        def outer(i, _):
            kk = i * 2

            @pl.when(kk < n)
            def _():
                body(kk, 0)

            @pl.when(kk + 1 < n)
            def _():
                body(kk + 1, 1)

            return 0

        lax.fori_loop(0, _outer_iters(num_chunks), outer, 0)
        wait_out(0)
        wait_out(1)

    return k


def kernel(indices_list, f0, f1, f2):
    nnz = indices_list.shape[1]
    ndim = f0.shape[0]
    idx = indices_list.astype(jnp.int32)
    p = _mul_kernel(nnz, ndim)(idx[0], idx[1], f0, f1)
    return _reduce_kernel(nnz, ndim)(idx[2], f2, p)
